# baseline (speedup 1.0000x reference)
import functools

import jax
import jax.numpy as jnp
from jax import lax
from jax.experimental import pallas as pl
from jax.experimental.pallas import tpu as pltpu
from jax.experimental.pallas import tpu_sc as plsc

D = 64
SEQ = 200
L = 16
NV = D // L
NC = 2
NS = 16
NW = NC * NS
CH = 128

_LANES = None


def _hsum16(v):
    lanes = lax.iota(jnp.int32, L)
    dnums = lax.GatherDimensionNumbers(
        offset_dims=(), collapsed_slice_dims=(0,), start_index_map=(0,))
    for sh in (8, 4, 2, 1):
        perm = lax.bitwise_xor(lanes, jnp.int32(sh))
        v = v + lax.gather(v, perm[:, None], dnums, slice_sizes=(1,),
                           mode=lax.GatherScatterMode.PROMISE_IN_BOUNDS)
    return v


def _bcast_lane(v, lane):
    dnums = lax.GatherDimensionNumbers(
        offset_dims=(), collapsed_slice_dims=(0,), start_index_map=(0,))
    idx = jnp.full((L,), lane, dtype=jnp.int32)
    return lax.gather(v, idx[:, None], dnums, slice_sizes=(1,),
                      mode=lax.GatherScatterMode.PROMISE_IN_BOUNDS)


def _rsqrt16(v):
    i = lax.bitcast_convert_type(v, jnp.int32)
    i = jnp.int32(0x5F3759DF) - lax.shift_right_logical(i, 1)
    y = lax.bitcast_convert_type(i, jnp.float32)
    for _ in range(2):
        y = y * (1.5 - 0.5 * v * y * y)
    return y


@functools.lru_cache(maxsize=None)
def _make_sc_embed(n_rows):
    rows_per_w = n_rows // NW
    ch_per_w = rows_per_w // CH
    mesh = plsc.VectorSubcoreMesh(core_axis_name="c", subcore_axis_name="s")

    @functools.partial(
        pl.kernel,
        out_type=jax.ShapeDtypeStruct((n_rows // 2, 2 * D), jnp.float32),
        mesh=mesh,
        scratch_types=[
            pltpu.VMEM((rows_per_w + L,), jnp.int32),
            pltpu.VMEM((CH,), jnp.int32),
            pltpu.VMEM((104, 128), jnp.float32),
            pltpu.VMEM((D,), jnp.float32),
            pltpu.VMEM((D,), jnp.float32),
            pltpu.VMEM((CH, 2 * D), jnp.float32),
            pltpu.VMEM((CH // 2, 2 * D), jnp.float32),
            pltpu.SemaphoreType.DMA,
        ],
    )
    def sc_embed(idx_hbm, wt_hbm, pos_hbm, sc_hbm, sh_hbm, out_hbm,
                 idx_v, pidx_v, pos_v, scale_v, shift_v, pairs_v, out_v, sem):
        wid = lax.axis_index("s") * NC + lax.axis_index("c")
        row0 = wid * rows_per_w
        pltpu.sync_copy(idx_hbm.at[pl.ds(row0, rows_per_w)],
                        idx_v.at[pl.ds(0, rows_per_w)])
        pltpu.sync_copy(pos_hbm.at[pl.ds(0, 104)], pos_v)
        pltpu.sync_copy(sc_hbm, scale_v)
        pltpu.sync_copy(sh_hbm, shift_v)

        scs = [scale_v[pl.ds(q * L, L)] for q in range(NV)]
        shs = [shift_v[pl.ds(q * L, L)] for q in range(NV)]
        lanes = lax.iota(jnp.int32, L)

        def chunk(t, carry):
            cb = t * CH

            @plsc.parallel_loop(0, CH // L, step=1, unroll=4)
            def _pidx(g):
                iv = idx_v[pl.ds(cb + g * L, L)]
                pidx_v[pl.ds(g * L, L)] = iv >> 1

            pltpu.async_copy(wt_hbm.at[pidx_v], pairs_v, sem).wait()

            @plsc.parallel_loop(0, CH, step=1, unroll=8)
            def _row(j):
                h = (idx_v[pl.ds(cb + j, L)][0] & 1) << 6
                p = (row0 + cb + j) % SEQ
                pr = p >> 1
                pc = (p & 1) << 6
                e = [pairs_v[j, pl.ds(h + q * L, L)]
                     + pos_v[pr, pl.ds(pc + q * L, L)]
                     for q in range(NV)]
                s = (e[0] + e[1]) + (e[2] + e[3])
                q2 = ((e[0] * e[0] + e[1] * e[1])
                      + (e[2] * e[2] + e[3] * e[3]))
                mu = _hsum16(s) * (1.0 / D)
                exx = _hsum16(q2) * (1.0 / D)
                y = _rsqrt16(exx - mu * mu + 1e-12)
                orow = j >> 1
                ocol = (j & 1) * D
                for q in range(NV):
                    out_v[orow, pl.ds(ocol + q * L, L)] = (
                        ((e[q] - mu) * y) * scs[q] + shs[q])

            pltpu.sync_copy(
                out_v,
                out_hbm.at[pl.ds(pl.multiple_of((row0 + cb) // 2, 8), CH // 2)])
            return carry

        lax.fori_loop(0, ch_per_w, chunk, 0)

    return sc_embed


def kernel(input_ids, word_table, pos_table, scale, shift):
    B, S = input_ids.shape
    assert S == SEQ and word_table.shape[1] == D
    idx = input_ids.reshape(-1).astype(jnp.int32)
    wt2 = word_table.reshape(-1, 2 * D)
    pos2 = pos_table.reshape(-1, 2 * D)
    out = _make_sc_embed(B * S)(idx, wt2, pos2, scale, shift)
    return out.reshape(B, S, D)
